# Initial kernel scaffold; baseline (speedup 1.0000x reference)
import jax
import jax.numpy as jnp
from jax.experimental import pallas as pl


def kernel(q_pts, s_pts, x, neighbor_idxs, kernel_points, weights):
    raise NotImplementedError("write your pallas kernel here")


# speedup vs baseline: 1.3130x; 1.3130x over previous
import functools

import jax
import jax.numpy as jnp
from jax import lax
from jax.experimental import pallas as pl
from jax.experimental.pallas import tpu as pltpu
from jax.experimental.pallas import tpu_sc as plsc

N = 10000
K = 32
D = 128
OUT = 128
P = 16
KP_EXTENT = 0.05

E = N * K
IDX_ROWS = E // 128
NW = 32
ROWS_PER_W = IDX_ROWS // NW
TAIL_ROWS = IDX_ROWS - NW * ROWS_PER_W
GR = 3
GROUPS = ROWS_PER_W // GR
CHUNK = GR * 128

B = 200
GRID = N // B


_MESH = plsc.VectorSubcoreMesh(
    core_axis_name="c", subcore_axis_name="s", num_cores=2, num_subcores=16
)


def _worker_loop(body):
    wid = lax.axis_index("s") * 2 + lax.axis_index("c")
    wbase = wid * ROWS_PER_W

    def g_body(g, carry):
        body(wbase + g * GR, GR)
        return carry

    lax.fori_loop(0, GROUPS, g_body, 0)

    @pl.when(wid < TAIL_ROWS)
    def _tail():
        body(NW * ROWS_PER_W + wid, 1)


def _sc_gather_features(idx1d, ftab):
    @functools.partial(
        pl.kernel,
        out_type=jax.ShapeDtypeStruct((E, D), jnp.float32),
        mesh=_MESH,
        scratch_types=[
            pltpu.VMEM((CHUNK,), jnp.int32),
            pltpu.VMEM((CHUNK, D), jnp.float32),
            pltpu.SemaphoreType.DMA,
        ],
    )
    def k(idx_hbm, ftab_hbm, nx_hbm, idx_v, rows_v, sem):
        def do_chunk(r0, nrows):
            e0 = r0 * 128
            nedge = nrows * 128
            pltpu.sync_copy(idx_hbm.at[pl.ds(e0, nedge)],
                            idx_v.at[pl.ds(0, nedge)])
            descs = []
            for j in range(nrows):
                descs.append(pltpu.async_copy(
                    ftab_hbm.at[idx_v.at[pl.ds(j * 128, 128)]],
                    rows_v.at[pl.ds(j * 128, 128)], sem))
            for dsc in descs:
                dsc.wait()
            pltpu.sync_copy(rows_v.at[pl.ds(0, nedge)],
                            nx_hbm.at[pl.ds(e0, nedge)])

        _worker_loop(do_chunk)

    return k(idx1d, ftab)


def _sc_gather_positions(idx1d, ptab):
    @functools.partial(
        pl.kernel,
        out_type=(
            jax.ShapeDtypeStruct((E,), jnp.float32),
            jax.ShapeDtypeStruct((E,), jnp.float32),
            jax.ShapeDtypeStruct((E,), jnp.float32),
        ),
        mesh=_MESH,
        scratch_types=[
            pltpu.VMEM((CHUNK,), jnp.int32),
            pltpu.VMEM((CHUNK, 16), jnp.float32),
            pltpu.VMEM((CHUNK,), jnp.float32),
            pltpu.VMEM((CHUNK,), jnp.float32),
            pltpu.VMEM((CHUNK,), jnp.float32),
            pltpu.SemaphoreType.DMA,
        ],
        compiler_params=pltpu.CompilerParams(
            needs_layout_passes=False,
            use_tc_tiling_on_sc=False,
        ),
    )
    def k(idx_hbm, ptab_hbm, px_hbm, py_hbm, pz_hbm,
          idx_v, prow_v, pcx_v, pcy_v, pcz_v, sem):
        pc = (pcx_v, pcy_v, pcz_v)
        phbm = (px_hbm, py_hbm, pz_hbm)

        def do_chunk(r0, nrows):
            e0 = r0 * 128
            nedge = nrows * 128
            pltpu.sync_copy(idx_hbm.at[pl.ds(e0, nedge)],
                            idx_v.at[pl.ds(0, nedge)])
            descs = []
            for j in range(nrows):
                descs.append(pltpu.async_copy(
                    ptab_hbm.at[idx_v.at[pl.ds(j * 128, 128)]],
                    prow_v.at[pl.ds(j * 128, 128)], sem))
            for dsc in descs:
                dsc.wait()
            base_iota = lax.iota(jnp.int32, 16)

            def tr_body(g, carry):
                rows = g * 16 + base_iota
                for c in range(3):
                    v = plsc.load_gather(
                        prow_v, [rows, jnp.full((16,), c, jnp.int32)])
                    pc[c][pl.ds(g * 16, 16)] = v
                return carry

            lax.fori_loop(0, nedge // 16, tr_body, 0)
            for c in range(3):
                pltpu.sync_copy(pc[c].at[pl.ds(0, nedge)],
                                phbm[c].at[pl.ds(e0, nedge)])

        _worker_loop(do_chunk)

    return k(idx1d, ptab)


def _tc_body(nx_ref, px_ref, py_ref, pz_ref, q_ref, kp_ref, w_ref, o_ref):
    qx = q_ref[:, 0:1]
    qy = q_ref[:, 1:2]
    qz = q_ref[:, 2:3]
    dx = px_ref[...] - qx
    dy = py_ref[...] - qy
    dz = pz_ref[...] - qz

    acc = jnp.zeros((B, OUT), jnp.float32)
    cnt = jnp.zeros((B, 1), jnp.float32)
    for k in range(K):
        s = jnp.sum(nx_ref[:, k, :], axis=1, keepdims=True)
        cnt += (s > 0.0).astype(jnp.float32)

    for p in range(P):
        ex = dx - kp_ref[p, 0]
        ey = dy - kp_ref[p, 1]
        ez = dz - kp_ref[p, 2]
        sq = ex * ex + ey * ey + ez * ez
        w = jnp.maximum(1.0 - jnp.sqrt(sq) * (1.0 / KP_EXTENT), 0.0)
        wf = jnp.zeros((B, D), jnp.float32)
        for k in range(K):
            wf += w[:, k:k + 1] * nx_ref[:, k, :]
        acc += jnp.dot(wf, w_ref[p], preferred_element_type=jnp.float32)

    inv = 1.0 / jnp.maximum(cnt, 1.0)
    o_ref[...] = acc * inv


def kernel(q_pts, s_pts, x, neighbor_idxs, kernel_points, weights):
    idx1d = neighbor_idxs.astype(jnp.int32).reshape(E)
    ftab = jnp.concatenate([x, jnp.zeros((1, D), jnp.float32)], axis=0)
    ptab = jnp.pad(s_pts, ((0, 1), (0, 13)))
    kp_pad = jnp.pad(kernel_points, ((0, 1), (0, 5)))
    w_pad = jnp.pad(weights, ((0, 1), (0, 0), (0, 0)))

    nx_flat = _sc_gather_features(idx1d, ftab)
    px, py, pz = _sc_gather_positions(idx1d, ptab)
    nx = nx_flat.reshape(N, K, D)
    px = px.reshape(N, K)
    py = py.reshape(N, K)
    pz = pz.reshape(N, K)

    out = pl.pallas_call(
        _tc_body,
        grid=(GRID,),
        in_specs=[
            pl.BlockSpec((B, K, D), lambda i: (i, 0, 0)),
            pl.BlockSpec((B, K), lambda i: (i, 0)),
            pl.BlockSpec((B, K), lambda i: (i, 0)),
            pl.BlockSpec((B, K), lambda i: (i, 0)),
            pl.BlockSpec((B, 3), lambda i: (i, 0)),
            pl.BlockSpec(memory_space=pltpu.SMEM),
            pl.BlockSpec((P, D, OUT), lambda i: (0, 0, 0)),
        ],
        out_specs=pl.BlockSpec((B, OUT), lambda i: (i, 0)),
        out_shape=jax.ShapeDtypeStruct((N, OUT), jnp.float32),
        compiler_params=pltpu.CompilerParams(
            dimension_semantics=("arbitrary",),
        ),
    )(nx, px, py, pz, q_pts, kp_pad, w_pad)
    return out
